# baseline (speedup 1.0000x reference)
import functools

import jax
import jax.numpy as jnp
from jax import lax
from jax.experimental import pallas as pl
from jax.experimental.pallas import tpu as pltpu
from jax.experimental.pallas import tpu_sc as plsc


_B, _S, _D = 4, 8192, 768
_NC, _NS = 2, 16
_NW = _NC * _NS
_ROWS_W = _S // _NW
_CHUNK = 64
_NCHUNK = _ROWS_W // _CHUNK
_CW = _CHUNK * _D
_NVEC = _CW // 16


def _sc_body(x_hbm, pos_hbm, out_hbm, xbuf, pbuf):
    wid = lax.axis_index("s") * _NC + lax.axis_index("c")
    pos_base = wid * (_ROWS_W * _D)

    def chunk_body(ci, carry):
        poff = pos_base + ci * _CW
        pltpu.sync_copy(pos_hbm.at[pl.ds(poff, _CW)], pbuf)

        def batch_body(b, carry):
            xoff = b * (_S * _D) + poff
            pltpu.sync_copy(x_hbm.at[pl.ds(xoff, _CW)], xbuf)

            def vec_body(i, carry):
                sl = pl.ds(i * 16, 16)
                plsc.addupdate(xbuf.at[sl], pbuf[sl])
                return carry

            lax.fori_loop(0, _NVEC, vec_body, 0)
            pltpu.sync_copy(xbuf, out_hbm.at[pl.ds(xoff, _CW)])
            return carry

        lax.fori_loop(0, _B, batch_body, 0)
        return carry

    lax.fori_loop(0, _NCHUNK, chunk_body, 0)


_sc_add = functools.partial(
    pl.kernel,
    mesh=plsc.VectorSubcoreMesh(core_axis_name="c", subcore_axis_name="s"),
    out_type=jax.ShapeDtypeStruct((_B * _S * _D,), jnp.float32),
    scratch_types=[
        pltpu.VMEM((_CW,), jnp.float32),
        pltpu.VMEM((_CW,), jnp.float32),
    ],
)(_sc_body)


def kernel(x, pos_table):
    out = _sc_add(x.reshape(-1), pos_table.reshape(-1))
    return out.reshape(x.shape)


# speedup vs baseline: 1.5828x; 1.5828x over previous
import functools

import jax
import jax.numpy as jnp
from jax import lax
from jax.experimental import pallas as pl
from jax.experimental.pallas import tpu as pltpu
from jax.experimental.pallas import tpu_sc as plsc


_B, _S, _D = 4, 8192, 768
_NC, _NS = 2, 16
_NW = _NC * _NS
_ROWS_W = _S // _NW
_CHUNK = 32
_CW = _CHUNK * _D
_NVEC = _CW // 16
_NPOS = _ROWS_W // _CHUNK
_NT = _NPOS * _B


def _sc_body(x_hbm, pos_hbm, out_hbm,
             xb0, xb1, pb0, pb1,
             sin0, sin1, sout0, sout1, spos0, spos1):
    wid = lax.axis_index("s") * _NC + lax.axis_index("c")
    pos_base = wid * (_ROWS_W * _D)
    xbufs, sins, souts = (xb0, xb1), (sin0, sin1), (sout0, sout1)
    pbufs, sposs = (pb0, pb1), (spos0, spos1)

    def x_off(t):
        ci, b = divmod(t, _B)
        return b * (_S * _D) + pos_base + ci * _CW

    def in_copy(t):
        return pltpu.make_async_copy(
            x_hbm.at[pl.ds(x_off(t), _CW)], xbufs[t % 2], sins[t % 2])

    def out_copy(t):
        return pltpu.make_async_copy(
            xbufs[t % 2], out_hbm.at[pl.ds(x_off(t), _CW)], souts[t % 2])

    def pos_copy(ci):
        return pltpu.make_async_copy(
            pos_hbm.at[pl.ds(pos_base + ci * _CW, _CW)],
            pbufs[ci % 2], sposs[ci % 2])

    pos_copy(0).start()
    in_copy(0).start()
    for t in range(_NT):
        ci, b = divmod(t, _B)
        if b == 0:
            pos_copy(ci).wait()
            if ci + 1 < _NPOS:
                pos_copy(ci + 1).start()
        if t + 1 < _NT:
            if t - 1 >= 0:
                out_copy(t - 1).wait()
            in_copy(t + 1).start()
        in_copy(t).wait()

        xb, pb = xbufs[t % 2], pbufs[ci % 2]

        @plsc.parallel_loop(0, _NVEC, unroll=8)
        def _(i):
            sl = pl.ds(i * 16, 16)
            plsc.addupdate(xb.at[sl], pb[sl])

        out_copy(t).start()
    out_copy(_NT - 2).wait()
    out_copy(_NT - 1).wait()


_sc_add = functools.partial(
    pl.kernel,
    mesh=plsc.VectorSubcoreMesh(core_axis_name="c", subcore_axis_name="s"),
    out_type=jax.ShapeDtypeStruct((_B * _S * _D,), jnp.float32),
    scratch_types=[
        pltpu.VMEM((_CW,), jnp.float32),
        pltpu.VMEM((_CW,), jnp.float32),
        pltpu.VMEM((_CW,), jnp.float32),
        pltpu.VMEM((_CW,), jnp.float32),
        pltpu.SemaphoreType.DMA,
        pltpu.SemaphoreType.DMA,
        pltpu.SemaphoreType.DMA,
        pltpu.SemaphoreType.DMA,
        pltpu.SemaphoreType.DMA,
        pltpu.SemaphoreType.DMA,
    ],
)(_sc_body)


def kernel(x, pos_table):
    out = _sc_add(x.reshape(-1), pos_table.reshape(-1))
    return out.reshape(x.shape)


# speedup vs baseline: 4.1485x; 2.6210x over previous
import functools

import jax
import jax.numpy as jnp
from jax import lax
from jax.experimental import pallas as pl
from jax.experimental.pallas import tpu as pltpu
from jax.experimental.pallas import tpu_sc as plsc


_B, _S, _D = 4, 8192, 768
_NC, _NS = 2, 16
_NW = _NC * _NS
_ROWS_W = _S // _NW
_CHUNK = 32
_NVPR = _D // 16
_NPOS = _ROWS_W // _CHUNK
_NCI2 = _NPOS // 2


def _sc_body(x_hbm, pos_hbm, out_hbm,
             xb0, xb1, pb0, pb1,
             sin0, sin1, sout0, sout1, spos0, spos1):
    wid = lax.axis_index("s") * _NC + lax.axis_index("c")
    row_base = wid * _ROWS_W
    xbufs, sins, souts = (xb0, xb1), (sin0, sin1), (sout0, sout1)
    pbufs, sposs = (pb0, pb1), (spos0, spos1)

    def rows(ci):
        return pl.ds(row_base + ci * _CHUNK, _CHUNK)

    def in_copy(ci, b):
        return pltpu.make_async_copy(
            x_hbm.at[b, rows(ci)], xbufs[b % 2], sins[b % 2])

    def out_copy(ci, b):
        return pltpu.make_async_copy(
            xbufs[b % 2], out_hbm.at[b, rows(ci)], souts[b % 2])

    def pos_copy(ci, q):
        return pltpu.make_async_copy(pos_hbm.at[rows(ci)], pbufs[q], sposs[q])

    def add_chunk(q, b):
        xb, pb = xbufs[b % 2], pbufs[q]

        @plsc.parallel_loop(0, _CHUNK, unroll=2)
        def _(r):
            for j in range(_NVPR):
                sl = pl.ds(j * 16, 16)
                plsc.addupdate(xb.at[r, sl], pb[r, sl])

    pos_copy(0, 0).start()
    in_copy(0, 0).start()

    def outer(ci2, carry):
        for q in range(2):
            ci = ci2 * 2 + q
            pos_copy(ci, q).wait()
            if q == 0:
                pos_copy(ci + 1, 1).start()
            else:
                @pl.when(ci2 < _NCI2 - 1)
                def _():
                    pos_copy(ci + 1, 0).start()
            for b in range(_B):
                if b == 0 and q == 0:
                    @pl.when(ci2 > 0)
                    def _():
                        out_copy(ci - 1, _B - 1).wait()

                    in_copy(ci, 1).start()
                elif b == 0:
                    out_copy(ci - 1, _B - 1).wait()
                    in_copy(ci, 1).start()
                elif b < _B - 1:
                    out_copy(ci, b - 1).wait()
                    in_copy(ci, b + 1).start()
                else:
                    out_copy(ci, b - 1).wait()
                    if q == 0:
                        in_copy(ci + 1, 0).start()
                    else:
                        @pl.when(ci2 < _NCI2 - 1)
                        def _():
                            in_copy(ci + 1, 0).start()
                in_copy(ci, b).wait()
                add_chunk(q, b)
                out_copy(ci, b).start()
        return carry

    lax.fori_loop(0, _NCI2, outer, 0)
    out_copy(_NPOS - 1, _B - 1).wait()


_sc_add = functools.partial(
    pl.kernel,
    mesh=plsc.VectorSubcoreMesh(core_axis_name="c", subcore_axis_name="s"),
    out_type=jax.ShapeDtypeStruct((_B, _S, _D), jnp.float32),
    scratch_types=[
        pltpu.VMEM((_CHUNK, _D), jnp.float32),
        pltpu.VMEM((_CHUNK, _D), jnp.float32),
        pltpu.VMEM((_CHUNK, _D), jnp.float32),
        pltpu.VMEM((_CHUNK, _D), jnp.float32),
        pltpu.SemaphoreType.DMA,
        pltpu.SemaphoreType.DMA,
        pltpu.SemaphoreType.DMA,
        pltpu.SemaphoreType.DMA,
        pltpu.SemaphoreType.DMA,
        pltpu.SemaphoreType.DMA,
    ],
)(_sc_body)


def kernel(x, pos_table):
    return _sc_add(x, pos_table)
